# baseline (speedup 1.0000x reference)
import functools

import jax
import jax.numpy as jnp
from jax import lax
from jax.experimental import pallas as pl
from jax.experimental.pallas import tpu as pltpu
from jax.experimental.pallas import tpu_sc as plsc

N_NODES = 10000
N_EDGES = 160000
DIM = 256
HALF = 128

NC = 2
NS = 16
LANES = 16

NPAD = 10240
CHUNK = 128
GRP = 8
EPT = 10240
NCHUNK = EPT // CHUNK
NGRP = NCHUNK // GRP
ROWS_PER_TILE = NPAD // NS
DEG_ROWS = NPAD // HALF
RECIP_TILES = 10
RECIP_ROWS = 8


def _sc_aggregate(table, src2, dst3, iota):
  mesh = plsc.VectorSubcoreMesh(
      core_axis_name="c", subcore_axis_name="s", num_cores=NC,
      num_subcores=NS)

  @functools.partial(
      pl.kernel,
      out_type=[
          jax.ShapeDtypeStruct((NPAD, HALF), jnp.float32),
          jax.ShapeDtypeStruct((NPAD, HALF), jnp.float32),
          jax.ShapeDtypeStruct((DEG_ROWS, HALF), jnp.float32),
      ],
      mesh=mesh,
      compiler_params=pltpu.CompilerParams(needs_layout_passes=False),
      scratch_types=[
          pltpu.VMEM((2, GRP, CHUNK), jnp.int32),
          pltpu.VMEM((2, GRP, CHUNK), jnp.int32),
          pltpu.VMEM((2, CHUNK, HALF), jnp.float32),
          pltpu.VMEM((DEG_ROWS, HALF), jnp.float32),
          pltpu.VMEM((DEG_ROWS,), jnp.int32),
          pltpu.VMEM((RECIP_ROWS, HALF), jnp.float32),
          pltpu.VMEM_SHARED((NPAD, HALF), jnp.float32),
          pltpu.VMEM_SHARED((DEG_ROWS, HALF), jnp.float32),
          pltpu.SemaphoreType.DMA,
          pltpu.SemaphoreType.DMA,
          pltpu.SemaphoreType.DMA,
          pltpu.SemaphoreType.DMA,
      ],
  )
  def sc_agg(table_h, src_h, dst_h, iota_h, summed0_out, summed1_out,
             recip_out, sbuf, dbuf, rows_v, deg_v, iota_v, dacc_v, accum,
             sdeg, sem_g0, sem_g1, sem_s, sem_i):
    c = lax.axis_index("c")
    s = lax.axis_index("s")
    zeros16 = jnp.zeros((LANES,), jnp.float32)
    ones16 = jnp.ones((LANES,), jnp.float32)

    pltpu.sync_copy(iota_h, iota_v)

    def zero_rows(i, _):
      rows_v[0, i // (HALF // LANES),
             pl.ds((i % (HALF // LANES)) * LANES, LANES)] = zeros16
      return 0
    lax.fori_loop(0, CHUNK * (HALF // LANES), zero_rows, 0, unroll=8)

    def zero_deg(i, _):
      deg_v[i // (HALF // LANES),
            pl.ds((i % (HALF // LANES)) * LANES, LANES)] = zeros16
      return 0
    lax.fori_loop(0, DEG_ROWS * (HALF // LANES), zero_deg, 0, unroll=8)

    for k in range(ROWS_PER_TILE // CHUNK):
      pltpu.sync_copy(rows_v.at[0],
                      accum.at[pl.ds(s * ROWS_PER_TILE + k * CHUNK, CHUNK)])
    @pl.when(s == 0)
    def _():
      pltpu.sync_copy(rows_v.at[0, pl.ds(0, DEG_ROWS)], sdeg)
    plsc.subcore_barrier()

    def idx_pf(g):
      base = s * NCHUNK + g * GRP
      return (pltpu.async_copy(src_h.at[pl.ds(base, GRP)], sbuf.at[g % 2],
                               sem_i),
              pltpu.async_copy(dst_h.at[c, pl.ds(base, GRP)], dbuf.at[g % 2],
                               sem_i))

    def gather(t):
      g, j = divmod(t, GRP)
      return pltpu.async_copy(table_h.at[dbuf.at[g % 2, j]], rows_v.at[t % 2],
                              sem_g0 if t % 2 == 0 else sem_g1)

    pf = idx_pf(0)
    pf[0].wait()
    pf[1].wait()
    gathers = {0: gather(0)}
    scatters = {}
    for t in range(NCHUNK):
      g, j = divmod(t, GRP)
      if t >= 1:
        scatters[t - 1].wait()
      if j == 0 and g + 1 < NGRP:
        pf = idx_pf(g + 1)
      if j == GRP - 1 and g + 1 < NGRP:
        pf[0].wait()
        pf[1].wait()
      if t + 1 < NCHUNK:
        gathers[t + 1] = gather(t + 1)
      gathers[t].wait()
      scatters[t] = pltpu.async_copy(rows_v.at[t % 2],
                                     accum.at[sbuf.at[g % 2, j]], sem_s,
                                     add=True)
      if j == GRP - 1:
        @pl.when(c == 0)
        def _(gp=g % 2):
          def deg_body(i, _):
            idx = sbuf[gp, i // (CHUNK // LANES),
                       pl.ds((i % (CHUNK // LANES)) * LANES, LANES)]
            row = lax.shift_right_logical(idx, 7)
            col = lax.bitwise_and(idx, HALF - 1)
            plsc.addupdate_scatter(deg_v, [row, col], ones16)
            return 0
          lax.fori_loop(0, GRP * (CHUNK // LANES), deg_body, 0, unroll=4)
    scatters[NCHUNK - 1].wait()

    @pl.when(c == 0)
    def _():
      pltpu.sync_copy(deg_v, sdeg.at[iota_v], add=True)
    plsc.subcore_barrier()

    @pl.when(c == 0)
    def _():
      pltpu.sync_copy(accum.at[pl.ds(s * ROWS_PER_TILE, ROWS_PER_TILE)],
                      summed0_out.at[pl.ds(s * ROWS_PER_TILE, ROWS_PER_TILE)])
    @pl.when(c == 1)
    def _():
      pltpu.sync_copy(accum.at[pl.ds(s * ROWS_PER_TILE, ROWS_PER_TILE)],
                      summed1_out.at[pl.ds(s * ROWS_PER_TILE, ROWS_PER_TILE)])

    @pl.when(jnp.logical_and(c == 0, s < RECIP_TILES))
    def _():
      pltpu.sync_copy(sdeg.at[pl.ds(s * RECIP_ROWS, RECIP_ROWS)], dacc_v)
      def recip_body(i, _):
        r = i // (HALF // LANES)
        k = (i % (HALF // LANES)) * LANES
        d = dacc_v[r, pl.ds(k, LANES)]
        dacc_v[r, pl.ds(k, LANES)] = 1.0 / jnp.maximum(d, 1.0)
        return 0
      lax.fori_loop(0, RECIP_ROWS * (HALF // LANES), recip_body, 0, unroll=8)
      pltpu.sync_copy(dacc_v, recip_out.at[pl.ds(s * RECIP_ROWS, RECIP_ROWS)])

  return sc_agg(table, src2, dst3, iota)


ROW_BLK = 1000
GRID = N_NODES // ROW_BLK


def _tc_self_body(h_ref, ws_ref, bs_ref, bn_ref, o_ref):
  dn = (((1,), (1,)), ((), ()))
  acc = lax.dot_general(h_ref[...], ws_ref[...], dn,
                        preferred_element_type=jnp.float32)
  o_ref[...] = acc + bs_ref[...] + bn_ref[...]


def _tc_self(h, W_self, b_self, b_neigh):
  return pl.pallas_call(
      _tc_self_body,
      grid=(GRID,),
      in_specs=[
          pl.BlockSpec((ROW_BLK, DIM), lambda i: (i, 0)),
          pl.BlockSpec((DIM, DIM), lambda i: (0, 0)),
          pl.BlockSpec((1, DIM), lambda i: (0, 0)),
          pl.BlockSpec((1, DIM), lambda i: (0, 0)),
      ],
      out_specs=pl.BlockSpec((ROW_BLK, DIM), lambda i: (i, 0)),
      out_shape=jax.ShapeDtypeStruct((N_NODES, DIM), jnp.float32),
  )(h, W_self, b_self, b_neigh)


def _tc_body(sa_ref, wn_ref, s0_ref, s1_ref, r_ref, o_ref):
  r = r_ref[...]
  n0 = s0_ref[...] * r
  n1 = s1_ref[...] * r
  dn = (((1,), (1,)), ((), ()))
  acc = sa_ref[...] + lax.dot_general(n0, wn_ref[:, 0:HALF], dn,
                                      preferred_element_type=jnp.float32)
  acc = acc + lax.dot_general(n1, wn_ref[:, HALF:DIM], dn,
                              preferred_element_type=jnp.float32)
  o_ref[...] = jnp.maximum(acc, 0.0)


def _tc_combine(selfacc, W_neigh, s0, s1, recip):
  return pl.pallas_call(
      _tc_body,
      grid=(GRID,),
      in_specs=[
          pl.BlockSpec((ROW_BLK, DIM), lambda i: (i, 0)),
          pl.BlockSpec((DIM, DIM), lambda i: (0, 0)),
          pl.BlockSpec((ROW_BLK, HALF), lambda i: (i, 0)),
          pl.BlockSpec((ROW_BLK, HALF), lambda i: (i, 0)),
          pl.BlockSpec((ROW_BLK, 1), lambda i: (i, 0)),
      ],
      out_specs=pl.BlockSpec((ROW_BLK, DIM), lambda i: (i, 0)),
      out_shape=jax.ShapeDtypeStruct((N_NODES, DIM), jnp.float32),
  )(selfacc, W_neigh, s0, s1, recip)


def kernel(embed_weight, W_self, b_self, W_neigh, b_neigh, edge_index):
  h = embed_weight.astype(jnp.float32)
  src = edge_index[0].astype(jnp.int32)
  dst = edge_index[1].astype(jnp.int32)

  e_pad = NS * EPT
  npad_rows = NPAD - N_NODES
  pad_src = N_NODES + jnp.arange(e_pad - N_EDGES, dtype=jnp.int32) % npad_rows
  pad_dst = jnp.arange(e_pad - N_EDGES, dtype=jnp.int32) % npad_rows
  src2 = jnp.concatenate([src, pad_src]).reshape(NS * NCHUNK, CHUNK)
  dstp = jnp.concatenate([dst, pad_dst]).reshape(NS * NCHUNK, CHUNK)
  table = jnp.concatenate([h[:, :HALF], h[:, HALF:]], axis=0)
  dst3 = jnp.stack([dstp, dstp + N_NODES])
  iota = jnp.arange(DEG_ROWS, dtype=jnp.int32)

  s0, s1, recip = _sc_aggregate(table, src2, dst3, iota)
  selfacc = _tc_self(h, W_self, b_self.reshape(1, DIM),
                     b_neigh.reshape(1, DIM))
  return _tc_combine(selfacc, W_neigh, s0, s1, recip.reshape(NPAD, 1))


# speedup vs baseline: 1.0052x; 1.0052x over previous
import functools

import jax
import jax.numpy as jnp
from jax import lax
from jax.experimental import pallas as pl
from jax.experimental.pallas import tpu as pltpu
from jax.experimental.pallas import tpu_sc as plsc

N_NODES = 10000
N_EDGES = 160000
DIM = 256
HALF = 128

NC = 2
NS = 16
LANES = 16

NPAD = 10240
CHUNK = 128
GRP = 8
EPT = 10240
NCHUNK = EPT // CHUNK
NGRP = NCHUNK // GRP
DEPTH = 2
AHEAD = 1
ROWS_PER_TILE = NPAD // NS
DEG_ROWS = NPAD // HALF
RECIP_TILES = 10
RECIP_ROWS = 8


def _sc_aggregate(table, src2, dst3, iota):
  mesh = plsc.VectorSubcoreMesh(
      core_axis_name="c", subcore_axis_name="s", num_cores=NC,
      num_subcores=NS)

  @functools.partial(
      pl.kernel,
      out_type=[
          jax.ShapeDtypeStruct((NPAD, HALF), jnp.float32),
          jax.ShapeDtypeStruct((NPAD, HALF), jnp.float32),
          jax.ShapeDtypeStruct((DEG_ROWS, HALF), jnp.float32),
      ],
      mesh=mesh,
      compiler_params=pltpu.CompilerParams(needs_layout_passes=False),
      scratch_types=[
          pltpu.VMEM((2, GRP, CHUNK), jnp.int32),
          pltpu.VMEM((2, GRP, CHUNK), jnp.int32),
          pltpu.VMEM((DEPTH, CHUNK, HALF), jnp.float32),
          pltpu.VMEM((DEG_ROWS, HALF), jnp.float32),
          pltpu.VMEM((DEG_ROWS,), jnp.int32),
          pltpu.VMEM((RECIP_ROWS, HALF), jnp.float32),
          pltpu.VMEM_SHARED((NPAD, HALF), jnp.float32),
          pltpu.VMEM_SHARED((DEG_ROWS, HALF), jnp.float32),
          pltpu.SemaphoreType.DMA,
          pltpu.SemaphoreType.DMA,
          pltpu.SemaphoreType.DMA,
          pltpu.SemaphoreType.DMA,
          pltpu.SemaphoreType.DMA,
      ],
  )
  def sc_agg(table_h, src_h, dst_h, iota_h, summed0_out, summed1_out,
             recip_out, sbuf, dbuf, rows_v, deg_v, iota_v, dacc_v, accum,
             sdeg, g0, g1, s0, s1, sem_i):
    sem_g = [g0, g1]
    sem_s = [s0, s1]
    c = lax.axis_index("c")
    s = lax.axis_index("s")
    zeros16 = jnp.zeros((LANES,), jnp.float32)
    ones16 = jnp.ones((LANES,), jnp.float32)

    pltpu.sync_copy(iota_h, iota_v)

    def zero_rows(i, _):
      rows_v[0, i // (HALF // LANES),
             pl.ds((i % (HALF // LANES)) * LANES, LANES)] = zeros16
      return 0
    lax.fori_loop(0, CHUNK * (HALF // LANES), zero_rows, 0, unroll=8)

    def zero_deg(i, _):
      deg_v[i // (HALF // LANES),
            pl.ds((i % (HALF // LANES)) * LANES, LANES)] = zeros16
      return 0
    lax.fori_loop(0, DEG_ROWS * (HALF // LANES), zero_deg, 0, unroll=8)

    for k in range(ROWS_PER_TILE // CHUNK):
      pltpu.sync_copy(rows_v.at[0],
                      accum.at[pl.ds(s * ROWS_PER_TILE + k * CHUNK, CHUNK)])
    @pl.when(s == 0)
    def _():
      pltpu.sync_copy(rows_v.at[0, pl.ds(0, DEG_ROWS)], sdeg)
    plsc.subcore_barrier()

    def idx_pf(g):
      base = s * NCHUNK + g * GRP
      return (pltpu.async_copy(src_h.at[pl.ds(base, GRP)], sbuf.at[g % 2],
                               sem_i),
              pltpu.async_copy(dst_h.at[c, pl.ds(base, GRP)], dbuf.at[g % 2],
                               sem_i))

    def gather(t):
      g, j = divmod(t, GRP)
      return pltpu.async_copy(table_h.at[dbuf.at[g % 2, j]],
                              rows_v.at[t % DEPTH], sem_g[t % DEPTH])

    def scatter(t):
      g, j = divmod(t, GRP)
      return pltpu.async_copy(rows_v.at[t % DEPTH],
                              accum.at[sbuf.at[g % 2, j]], sem_s[t % DEPTH],
                              add=True)

    pf = idx_pf(0)
    pf[0].wait()
    pf[1].wait()
    gathers = {}
    scatters = {}
    for x in range(AHEAD):
      gathers[x] = gather(x)
    for t in range(NCHUNK):
      g, j = divmod(t, GRP)
      x = t + AHEAD
      if x < NCHUNK and x >= DEPTH:
        scatters[x - DEPTH].wait()
      if j == 0 and g + 1 < NGRP:
        pf = idx_pf(g + 1)
      if j == GRP - AHEAD and g + 1 < NGRP:
        pf[0].wait()
        pf[1].wait()
      if x < NCHUNK:
        gathers[x] = gather(x)
      gathers[t].wait()
      scatters[t] = scatter(t)
      if j == GRP - 1:
        @pl.when(c == 0)
        def _(gp=g % 2):
          def deg_body(i, _):
            idx = sbuf[gp, i // (CHUNK // LANES),
                       pl.ds((i % (CHUNK // LANES)) * LANES, LANES)]
            row = lax.shift_right_logical(idx, 7)
            col = lax.bitwise_and(idx, HALF - 1)
            plsc.addupdate_scatter(deg_v, [row, col], ones16)
            return 0
          lax.fori_loop(0, GRP * (CHUNK // LANES), deg_body, 0, unroll=4)
    for t in range(NCHUNK - DEPTH, NCHUNK):
      scatters[t].wait()

    @pl.when(c == 0)
    def _():
      pltpu.sync_copy(deg_v, sdeg.at[iota_v], add=True)
    plsc.subcore_barrier()

    @pl.when(c == 0)
    def _():
      pltpu.sync_copy(accum.at[pl.ds(s * ROWS_PER_TILE, ROWS_PER_TILE)],
                      summed0_out.at[pl.ds(s * ROWS_PER_TILE, ROWS_PER_TILE)])
    @pl.when(c == 1)
    def _():
      pltpu.sync_copy(accum.at[pl.ds(s * ROWS_PER_TILE, ROWS_PER_TILE)],
                      summed1_out.at[pl.ds(s * ROWS_PER_TILE, ROWS_PER_TILE)])

    @pl.when(jnp.logical_and(c == 0, s < RECIP_TILES))
    def _():
      pltpu.sync_copy(sdeg.at[pl.ds(s * RECIP_ROWS, RECIP_ROWS)], dacc_v)
      def recip_body(i, _):
        r = i // (HALF // LANES)
        k = (i % (HALF // LANES)) * LANES
        d = dacc_v[r, pl.ds(k, LANES)]
        dacc_v[r, pl.ds(k, LANES)] = 1.0 / jnp.maximum(d, 1.0)
        return 0
      lax.fori_loop(0, RECIP_ROWS * (HALF // LANES), recip_body, 0, unroll=8)
      pltpu.sync_copy(dacc_v, recip_out.at[pl.ds(s * RECIP_ROWS, RECIP_ROWS)])

  return sc_agg(table, src2, dst3, iota)


ROW_BLK = 1000
GRID = N_NODES // ROW_BLK


def _tc_body(h_ref, ws_ref, wn_ref, bs_ref, bn_ref, s0_ref, s1_ref, r_ref,
             o_ref):
  bf = jnp.bfloat16
  r = r_ref[...]
  n0 = (s0_ref[...] * r).astype(bf)
  n1 = (s1_ref[...] * r).astype(bf)
  dn = (((1,), (1,)), ((), ()))
  acc = lax.dot_general(h_ref[...].astype(bf), ws_ref[...].astype(bf), dn,
                        preferred_element_type=jnp.float32)
  acc = acc + lax.dot_general(n0, wn_ref[:, 0:HALF].astype(bf), dn,
                              preferred_element_type=jnp.float32)
  acc = acc + lax.dot_general(n1, wn_ref[:, HALF:DIM].astype(bf), dn,
                              preferred_element_type=jnp.float32)
  o_ref[...] = jnp.maximum(acc + bs_ref[...] + bn_ref[...], 0.0)


def _tc_combine(h, W_self, W_neigh, b_self, b_neigh, s0, s1, recip):
  return pl.pallas_call(
      _tc_body,
      grid=(GRID,),
      in_specs=[
          pl.BlockSpec((ROW_BLK, DIM), lambda i: (i, 0)),
          pl.BlockSpec((DIM, DIM), lambda i: (0, 0)),
          pl.BlockSpec((DIM, DIM), lambda i: (0, 0)),
          pl.BlockSpec((1, DIM), lambda i: (0, 0)),
          pl.BlockSpec((1, DIM), lambda i: (0, 0)),
          pl.BlockSpec((ROW_BLK, HALF), lambda i: (i, 0)),
          pl.BlockSpec((ROW_BLK, HALF), lambda i: (i, 0)),
          pl.BlockSpec((ROW_BLK, 1), lambda i: (i, 0)),
      ],
      out_specs=pl.BlockSpec((ROW_BLK, DIM), lambda i: (i, 0)),
      out_shape=jax.ShapeDtypeStruct((N_NODES, DIM), jnp.float32),
  )(h, W_self, W_neigh, b_self, b_neigh, s0, s1, recip)


def kernel(embed_weight, W_self, b_self, W_neigh, b_neigh, edge_index):
  h = embed_weight.astype(jnp.float32)
  src = edge_index[0].astype(jnp.int32)
  dst = edge_index[1].astype(jnp.int32)

  e_pad = NS * EPT
  npad_rows = NPAD - N_NODES
  pad_src = N_NODES + jnp.arange(e_pad - N_EDGES, dtype=jnp.int32) % npad_rows
  pad_dst = jnp.arange(e_pad - N_EDGES, dtype=jnp.int32) % npad_rows
  src2 = jnp.concatenate([src, pad_src]).reshape(NS * NCHUNK, CHUNK)
  dstp = jnp.concatenate([dst, pad_dst]).reshape(NS * NCHUNK, CHUNK)
  table = jnp.concatenate([h[:, :HALF], h[:, HALF:]], axis=0)
  dst3 = jnp.stack([dstp, dstp + N_NODES])
  iota = jnp.arange(DEG_ROWS, dtype=jnp.int32)

  s0, s1, recip = _sc_aggregate(table, src2, dst3, iota)
  return _tc_combine(h, W_self, W_neigh, b_self.reshape(1, DIM),
                     b_neigh.reshape(1, DIM), s0, s1,
                     recip.reshape(NPAD, 1))


# speedup vs baseline: 1.0180x; 1.0128x over previous
import functools

import jax
import jax.numpy as jnp
from jax import lax
from jax.experimental import pallas as pl
from jax.experimental.pallas import tpu as pltpu
from jax.experimental.pallas import tpu_sc as plsc

N_NODES = 10000
N_EDGES = 160000
DIM = 256
HALF = 128

NC = 2
NS = 16
LANES = 16

NPAD = 10240
CHUNK = 128
GRP = 8
EPT = 10240
NCHUNK = EPT // CHUNK
NGRP = NCHUNK // GRP
DEPTH = 2
AHEAD = 1
ROWS_PER_TILE = NPAD // NS
DEG_ROWS = NPAD // HALF
RECIP_TILES = 10
RECIP_ROWS = 8


def _sc_aggregate(table, src2, dst3, iota):
  mesh = plsc.VectorSubcoreMesh(
      core_axis_name="c", subcore_axis_name="s", num_cores=NC,
      num_subcores=NS)

  @functools.partial(
      pl.kernel,
      out_type=[
          jax.ShapeDtypeStruct((NPAD, HALF), jnp.float32),
          jax.ShapeDtypeStruct((NPAD, HALF), jnp.float32),
          jax.ShapeDtypeStruct((DEG_ROWS, HALF), jnp.float32),
      ],
      mesh=mesh,
      compiler_params=pltpu.CompilerParams(needs_layout_passes=False),
      scratch_types=[
          pltpu.VMEM((2, GRP, CHUNK), jnp.int32),
          pltpu.VMEM((2, GRP, CHUNK), jnp.int32),
          pltpu.VMEM((DEPTH, CHUNK, HALF), jnp.float32),
          pltpu.VMEM((DEG_ROWS, HALF), jnp.float32),
          pltpu.VMEM((DEG_ROWS,), jnp.int32),
          pltpu.VMEM((RECIP_ROWS, HALF), jnp.float32),
          pltpu.VMEM_SHARED((NPAD, HALF), jnp.float32),
          pltpu.VMEM_SHARED((DEG_ROWS, HALF), jnp.float32),
          pltpu.SemaphoreType.DMA,
          pltpu.SemaphoreType.DMA,
          pltpu.SemaphoreType.DMA,
          pltpu.SemaphoreType.DMA,
          pltpu.SemaphoreType.DMA,
      ],
  )
  def sc_agg(table_h, src_h, dst_h, iota_h, summed0_out, summed1_out,
             recip_out, sbuf, dbuf, rows_v, deg_v, iota_v, dacc_v, accum,
             sdeg, g0, g1, s0, s1, sem_i):
    sem_g = [g0, g1]
    sem_s = [s0, s1]
    c = lax.axis_index("c")
    s = lax.axis_index("s")
    zeros16 = jnp.zeros((LANES,), jnp.float32)
    ones16 = jnp.ones((LANES,), jnp.float32)

    pltpu.sync_copy(iota_h, iota_v)

    def zero_rows(i, _):
      rows_v[0, i // (HALF // LANES),
             pl.ds((i % (HALF // LANES)) * LANES, LANES)] = zeros16
      return 0
    lax.fori_loop(0, CHUNK * (HALF // LANES), zero_rows, 0, unroll=8)

    def zero_deg(i, _):
      deg_v[i // (HALF // LANES),
            pl.ds((i % (HALF // LANES)) * LANES, LANES)] = zeros16
      return 0
    lax.fori_loop(0, DEG_ROWS * (HALF // LANES), zero_deg, 0, unroll=8)

    for k in range(ROWS_PER_TILE // CHUNK):
      pltpu.sync_copy(rows_v.at[0],
                      accum.at[pl.ds(s * ROWS_PER_TILE + k * CHUNK, CHUNK)])
    @pl.when(s == 0)
    def _():
      pltpu.sync_copy(rows_v.at[0, pl.ds(0, DEG_ROWS)], sdeg)
    plsc.subcore_barrier()

    def idx_pf(g):
      base = s * NCHUNK + g * GRP
      return (pltpu.async_copy(src_h.at[pl.ds(base, GRP)], sbuf.at[g % 2],
                               sem_i),
              pltpu.async_copy(dst_h.at[c, pl.ds(base, GRP)], dbuf.at[g % 2],
                               sem_i))

    def gather(t):
      g, j = divmod(t, GRP)
      return pltpu.async_copy(table_h.at[dbuf.at[g % 2, j]],
                              rows_v.at[t % DEPTH], sem_g[t % DEPTH])

    def scatter(t):
      g, j = divmod(t, GRP)
      return pltpu.async_copy(rows_v.at[t % DEPTH],
                              accum.at[sbuf.at[g % 2, j]], sem_s[t % DEPTH],
                              add=True)

    pf = idx_pf(0)
    pf[0].wait()
    pf[1].wait()
    gathers = {}
    scatters = {}
    for x in range(AHEAD):
      gathers[x] = gather(x)
    for t in range(NCHUNK):
      g, j = divmod(t, GRP)
      x = t + AHEAD
      if x < NCHUNK and x >= DEPTH:
        scatters[x - DEPTH].wait()
      if j == 0 and g + 1 < NGRP:
        pf = idx_pf(g + 1)
      if j == GRP - AHEAD and g + 1 < NGRP:
        pf[0].wait()
        pf[1].wait()
      if x < NCHUNK:
        gathers[x] = gather(x)
      gathers[t].wait()
      scatters[t] = scatter(t)
      if j == GRP - 1:
        @pl.when(c == 0)
        def _(gp=g % 2):
          def deg_body(i, _):
            idx = sbuf[gp, i // (CHUNK // LANES),
                       pl.ds((i % (CHUNK // LANES)) * LANES, LANES)]
            row = lax.shift_right_logical(idx, 7)
            col = lax.bitwise_and(idx, HALF - 1)
            plsc.addupdate_scatter(deg_v, [row, col], ones16)
            return 0
          lax.fori_loop(0, GRP * (CHUNK // LANES), deg_body, 0, unroll=4)
    for t in range(NCHUNK - DEPTH, NCHUNK):
      scatters[t].wait()

    @pl.when(c == 0)
    def _():
      pltpu.sync_copy(deg_v, sdeg.at[iota_v], add=True)
    plsc.subcore_barrier()

    @pl.when(c == 0)
    def _():
      pltpu.sync_copy(accum.at[pl.ds(s * ROWS_PER_TILE, ROWS_PER_TILE)],
                      summed0_out.at[pl.ds(s * ROWS_PER_TILE, ROWS_PER_TILE)])
    @pl.when(c == 1)
    def _():
      pltpu.sync_copy(accum.at[pl.ds(s * ROWS_PER_TILE, ROWS_PER_TILE)],
                      summed1_out.at[pl.ds(s * ROWS_PER_TILE, ROWS_PER_TILE)])

    @pl.when(jnp.logical_and(c == 0, s < RECIP_TILES))
    def _():
      pltpu.sync_copy(sdeg.at[pl.ds(s * RECIP_ROWS, RECIP_ROWS)], dacc_v)
      def recip_body(i, _):
        r = i // (HALF // LANES)
        k = (i % (HALF // LANES)) * LANES
        d = dacc_v[r, pl.ds(k, LANES)]
        dacc_v[r, pl.ds(k, LANES)] = 1.0 / jnp.maximum(d, 1.0)
        return 0
      lax.fori_loop(0, RECIP_ROWS * (HALF // LANES), recip_body, 0, unroll=8)
      pltpu.sync_copy(dacc_v, recip_out.at[pl.ds(s * RECIP_ROWS, RECIP_ROWS)])

  return sc_agg(table, src2, dst3, iota)


ROW_BLK = 2000
GRID = N_NODES // ROW_BLK


def _tc_body(h_ref, ws_ref, wn_ref, bs_ref, bn_ref, s0_ref, s1_ref, r_ref,
             o_ref):
  r = r_ref[...]
  n0 = s0_ref[...] * r
  n1 = s1_ref[...] * r
  dn = (((1,), (1,)), ((), ()))
  acc = lax.dot_general(h_ref[...], ws_ref[...], dn,
                        preferred_element_type=jnp.float32)
  acc = acc + lax.dot_general(n0, wn_ref[:, 0:HALF], dn,
                              preferred_element_type=jnp.float32)
  acc = acc + lax.dot_general(n1, wn_ref[:, HALF:DIM], dn,
                              preferred_element_type=jnp.float32)
  o_ref[...] = jnp.maximum(acc + bs_ref[...] + bn_ref[...], 0.0)


def _tc_combine(h, W_self, W_neigh, b_self, b_neigh, s0, s1, recip):
  return pl.pallas_call(
      _tc_body,
      grid=(GRID,),
      in_specs=[
          pl.BlockSpec((ROW_BLK, DIM), lambda i: (i, 0)),
          pl.BlockSpec((DIM, DIM), lambda i: (0, 0)),
          pl.BlockSpec((DIM, DIM), lambda i: (0, 0)),
          pl.BlockSpec((1, DIM), lambda i: (0, 0)),
          pl.BlockSpec((1, DIM), lambda i: (0, 0)),
          pl.BlockSpec((ROW_BLK, HALF), lambda i: (i, 0)),
          pl.BlockSpec((ROW_BLK, HALF), lambda i: (i, 0)),
          pl.BlockSpec((ROW_BLK, 1), lambda i: (i, 0)),
      ],
      out_specs=pl.BlockSpec((ROW_BLK, DIM), lambda i: (i, 0)),
      out_shape=jax.ShapeDtypeStruct((N_NODES, DIM), jnp.float32),
  )(h, W_self, W_neigh, b_self, b_neigh, s0, s1, recip)


def kernel(embed_weight, W_self, b_self, W_neigh, b_neigh, edge_index):
  h = embed_weight.astype(jnp.float32)
  src = edge_index[0].astype(jnp.int32)
  dst = edge_index[1].astype(jnp.int32)

  e_pad = NS * EPT
  npad_rows = NPAD - N_NODES
  pad_src = N_NODES + jnp.arange(e_pad - N_EDGES, dtype=jnp.int32) % npad_rows
  pad_dst = jnp.arange(e_pad - N_EDGES, dtype=jnp.int32) % npad_rows
  src2 = jnp.concatenate([src, pad_src]).reshape(NS * NCHUNK, CHUNK)
  dstp = jnp.concatenate([dst, pad_dst]).reshape(NS * NCHUNK, CHUNK)
  table = jnp.concatenate([h[:, :HALF], h[:, HALF:]], axis=0)
  dst3 = jnp.stack([dstp, dstp + N_NODES])
  iota = jnp.arange(DEG_ROWS, dtype=jnp.int32)

  s0, s1, recip = _sc_aggregate(table, src2, dst3, iota)
  return _tc_combine(h, W_self, W_neigh, b_self.reshape(1, DIM),
                     b_neigh.reshape(1, DIM), s0, s1,
                     recip.reshape(NPAD, 1))


# speedup vs baseline: 1.0334x; 1.0151x over previous
import functools

import jax
import jax.numpy as jnp
from jax import lax
from jax.experimental import pallas as pl
from jax.experimental.pallas import tpu as pltpu
from jax.experimental.pallas import tpu_sc as plsc

N_NODES = 10000
N_EDGES = 160000
DIM = 256
HALF = 128

NC = 2
NS = 16
LANES = 16

NPAD = 10240
CHUNK = 128
GRP = 8
EPT = 10240
NCHUNK = EPT // CHUNK
NGRP = NCHUNK // GRP
DEPTH = 2
AHEAD = 1
ROWS_PER_TILE = NPAD // NS
DEG_ROWS = NPAD // HALF
RECIP_TILES = 10
RECIP_ROWS = 8


def _sc_aggregate(table, src2, dst3, iota):
  mesh = plsc.VectorSubcoreMesh(
      core_axis_name="c", subcore_axis_name="s", num_cores=NC,
      num_subcores=NS)

  @functools.partial(
      pl.kernel,
      out_type=[
          jax.ShapeDtypeStruct((NPAD, HALF), jnp.float32),
          jax.ShapeDtypeStruct((NPAD, HALF), jnp.float32),
          jax.ShapeDtypeStruct((DEG_ROWS, HALF), jnp.float32),
      ],
      mesh=mesh,
      compiler_params=pltpu.CompilerParams(needs_layout_passes=False),
      scratch_types=[
          pltpu.VMEM((2, GRP, CHUNK), jnp.int32),
          pltpu.VMEM((2, GRP, CHUNK), jnp.int32),
          pltpu.VMEM((DEPTH, CHUNK, HALF), jnp.float32),
          pltpu.VMEM((DEG_ROWS, HALF), jnp.float32),
          pltpu.VMEM((DEG_ROWS,), jnp.int32),
          pltpu.VMEM((RECIP_ROWS, HALF), jnp.float32),
          pltpu.VMEM_SHARED((NPAD, HALF), jnp.float32),
          pltpu.VMEM_SHARED((DEG_ROWS, HALF), jnp.float32),
          pltpu.SemaphoreType.DMA,
          pltpu.SemaphoreType.DMA,
          pltpu.SemaphoreType.DMA,
          pltpu.SemaphoreType.DMA,
          pltpu.SemaphoreType.DMA,
      ],
  )
  def sc_agg(table_h, src_h, dst_h, iota_h, summed0_out, summed1_out,
             recip_out, sbuf, dbuf, rows_v, deg_v, iota_v, dacc_v, accum,
             sdeg, g0, g1, s0, s1, sem_i):
    sem_g = [g0, g1]
    sem_s = [s0, s1]
    c = lax.axis_index("c")
    s = lax.axis_index("s")
    zeros16 = jnp.zeros((LANES,), jnp.float32)
    ones16 = jnp.ones((LANES,), jnp.float32)

    pltpu.sync_copy(iota_h, iota_v)

    def idx_pf(g):
      base = s * NCHUNK + g * GRP
      return (pltpu.async_copy(src_h.at[pl.ds(base, GRP)], sbuf.at[g % 2],
                               sem_i),
              pltpu.async_copy(dst_h.at[c, pl.ds(base, GRP)], dbuf.at[g % 2],
                               sem_i))

    def gather(t):
      g, j = divmod(t, GRP)
      return pltpu.async_copy(table_h.at[dbuf.at[g % 2, j]],
                              rows_v.at[t % DEPTH], sem_g[t % DEPTH])

    def scatter(t):
      g, j = divmod(t, GRP)
      return pltpu.async_copy(rows_v.at[t % DEPTH],
                              accum.at[sbuf.at[g % 2, j]], sem_s[t % DEPTH],
                              add=True)

    pf = idx_pf(0)

    def zero_deg(i, _):
      deg_v[i // (HALF // LANES),
            pl.ds((i % (HALF // LANES)) * LANES, LANES)] = zeros16
      return 0
    lax.fori_loop(0, DEG_ROWS * (HALF // LANES), zero_deg, 0, unroll=8)

    pf[0].wait()
    pf[1].wait()
    gathers = {}
    scatters = {}
    for x in range(AHEAD):
      gathers[x] = gather(x)

    for k in range(ROWS_PER_TILE // DEG_ROWS):
      pltpu.sync_copy(deg_v,
                      accum.at[pl.ds(s * ROWS_PER_TILE + k * DEG_ROWS,
                                     DEG_ROWS)])
    @pl.when(s == 0)
    def _():
      pltpu.sync_copy(deg_v, sdeg)
    plsc.subcore_barrier()

    for t in range(NCHUNK):
      g, j = divmod(t, GRP)
      x = t + AHEAD
      if x < NCHUNK and x >= DEPTH:
        scatters[x - DEPTH].wait()
      if j == 0 and g + 1 < NGRP:
        pf = idx_pf(g + 1)
      if j == GRP - AHEAD and g + 1 < NGRP:
        pf[0].wait()
        pf[1].wait()
      if x < NCHUNK:
        gathers[x] = gather(x)
      gathers[t].wait()
      scatters[t] = scatter(t)
      if j == GRP - 1:
        @pl.when(c == 0)
        def _(gp=g % 2):
          def deg_body(i, _):
            idx = sbuf[gp, i // (CHUNK // LANES),
                       pl.ds((i % (CHUNK // LANES)) * LANES, LANES)]
            row = lax.shift_right_logical(idx, 7)
            col = lax.bitwise_and(idx, HALF - 1)
            plsc.addupdate_scatter(deg_v, [row, col], ones16)
            return 0
          lax.fori_loop(0, GRP * (CHUNK // LANES), deg_body, 0, unroll=4)
    for t in range(NCHUNK - DEPTH, NCHUNK):
      scatters[t].wait()

    @pl.when(c == 0)
    def _():
      pltpu.sync_copy(deg_v, sdeg.at[iota_v], add=True)
    plsc.subcore_barrier()

    @pl.when(c == 0)
    def _():
      pltpu.sync_copy(accum.at[pl.ds(s * ROWS_PER_TILE, ROWS_PER_TILE)],
                      summed0_out.at[pl.ds(s * ROWS_PER_TILE, ROWS_PER_TILE)])
    @pl.when(c == 1)
    def _():
      pltpu.sync_copy(accum.at[pl.ds(s * ROWS_PER_TILE, ROWS_PER_TILE)],
                      summed1_out.at[pl.ds(s * ROWS_PER_TILE, ROWS_PER_TILE)])

    @pl.when(jnp.logical_and(c == 0, s < RECIP_TILES))
    def _():
      pltpu.sync_copy(sdeg.at[pl.ds(s * RECIP_ROWS, RECIP_ROWS)], dacc_v)
      def recip_body(i, _):
        r = i // (HALF // LANES)
        k = (i % (HALF // LANES)) * LANES
        d = dacc_v[r, pl.ds(k, LANES)]
        dacc_v[r, pl.ds(k, LANES)] = 1.0 / jnp.maximum(d, 1.0)
        return 0
      lax.fori_loop(0, RECIP_ROWS * (HALF // LANES), recip_body, 0, unroll=8)
      pltpu.sync_copy(dacc_v, recip_out.at[pl.ds(s * RECIP_ROWS, RECIP_ROWS)])

  return sc_agg(table, src2, dst3, iota)


ROW_BLK = 2000
GRID = N_NODES // ROW_BLK


def _tc_body(h_ref, ws_ref, wn_ref, bs_ref, bn_ref, s0_ref, s1_ref, r_ref,
             o_ref):
  r = r_ref[...]
  n0 = s0_ref[...] * r
  n1 = s1_ref[...] * r
  dn = (((1,), (1,)), ((), ()))
  acc = lax.dot_general(h_ref[...], ws_ref[...], dn,
                        preferred_element_type=jnp.float32)
  acc = acc + lax.dot_general(n0, wn_ref[:, 0:HALF], dn,
                              preferred_element_type=jnp.float32)
  acc = acc + lax.dot_general(n1, wn_ref[:, HALF:DIM], dn,
                              preferred_element_type=jnp.float32)
  o_ref[...] = jnp.maximum(acc + bs_ref[...] + bn_ref[...], 0.0)


def _tc_combine(h, W_self, W_neigh, b_self, b_neigh, s0, s1, recip):
  return pl.pallas_call(
      _tc_body,
      grid=(GRID,),
      in_specs=[
          pl.BlockSpec((ROW_BLK, DIM), lambda i: (i, 0)),
          pl.BlockSpec((DIM, DIM), lambda i: (0, 0)),
          pl.BlockSpec((DIM, DIM), lambda i: (0, 0)),
          pl.BlockSpec((1, DIM), lambda i: (0, 0)),
          pl.BlockSpec((1, DIM), lambda i: (0, 0)),
          pl.BlockSpec((ROW_BLK, HALF), lambda i: (i, 0)),
          pl.BlockSpec((ROW_BLK, HALF), lambda i: (i, 0)),
          pl.BlockSpec((ROW_BLK, 1), lambda i: (i, 0)),
      ],
      out_specs=pl.BlockSpec((ROW_BLK, DIM), lambda i: (i, 0)),
      out_shape=jax.ShapeDtypeStruct((N_NODES, DIM), jnp.float32),
  )(h, W_self, W_neigh, b_self, b_neigh, s0, s1, recip)


def kernel(embed_weight, W_self, b_self, W_neigh, b_neigh, edge_index):
  h = embed_weight.astype(jnp.float32)
  src = edge_index[0].astype(jnp.int32)
  dst = edge_index[1].astype(jnp.int32)

  e_pad = NS * EPT
  npad_rows = NPAD - N_NODES
  pad_src = N_NODES + jnp.arange(e_pad - N_EDGES, dtype=jnp.int32) % npad_rows
  pad_dst = jnp.arange(e_pad - N_EDGES, dtype=jnp.int32) % npad_rows
  src2 = jnp.concatenate([src, pad_src]).reshape(NS * NCHUNK, CHUNK)
  dstp = jnp.concatenate([dst, pad_dst]).reshape(NS * NCHUNK, CHUNK)
  table = jnp.concatenate([h[:, :HALF], h[:, HALF:]], axis=0)
  dst3 = jnp.stack([dstp, dstp + N_NODES])
  iota = jnp.arange(DEG_ROWS, dtype=jnp.int32)

  s0, s1, recip = _sc_aggregate(table, src2, dst3, iota)
  return _tc_combine(h, W_self, W_neigh, b_self.reshape(1, DIM),
                     b_neigh.reshape(1, DIM), s0, s1,
                     recip.reshape(NPAD, 1))


# speedup vs baseline: 1.0453x; 1.0115x over previous
import functools

import jax
import jax.numpy as jnp
from jax import lax
from jax.experimental import pallas as pl
from jax.experimental.pallas import tpu as pltpu
from jax.experimental.pallas import tpu_sc as plsc

N_NODES = 10000
N_EDGES = 160000
DIM = 256
HALF = 128

NC = 2
NS = 16
LANES = 16

NPAD = 10240
CHUNK = 128
GRP = 8
EPT = 10240
NCHUNK = EPT // CHUNK
NGRP = NCHUNK // GRP
DEPTH = 2
AHEAD = 1
ROWS_PER_TILE = NPAD // NS
DEG_ROWS = NPAD // HALF
RECIP_TILES = 10
RECIP_ROWS = 8


def _sc_aggregate(table, src2, dst3, iota):
  mesh = plsc.VectorSubcoreMesh(
      core_axis_name="c", subcore_axis_name="s", num_cores=NC,
      num_subcores=NS)

  @functools.partial(
      pl.kernel,
      out_type=[
          jax.ShapeDtypeStruct((NPAD, HALF), jnp.float32),
          jax.ShapeDtypeStruct((NPAD, HALF), jnp.float32),
          jax.ShapeDtypeStruct((DEG_ROWS, HALF), jnp.float32),
      ],
      mesh=mesh,
      compiler_params=pltpu.CompilerParams(needs_layout_passes=False),
      scratch_types=[
          pltpu.VMEM((2, GRP, CHUNK), jnp.int32),
          pltpu.VMEM((2, GRP, CHUNK), jnp.int32),
          pltpu.VMEM((DEPTH, CHUNK, HALF), jnp.float32),
          pltpu.VMEM((DEG_ROWS, HALF), jnp.float32),
          pltpu.VMEM((DEG_ROWS,), jnp.int32),
          pltpu.VMEM((RECIP_ROWS, HALF), jnp.float32),
          pltpu.VMEM_SHARED((NPAD, HALF), jnp.float32),
          pltpu.VMEM_SHARED((DEG_ROWS, HALF), jnp.float32),
          pltpu.SemaphoreType.DMA,
          pltpu.SemaphoreType.DMA,
          pltpu.SemaphoreType.DMA,
          pltpu.SemaphoreType.DMA,
          pltpu.SemaphoreType.DMA,
      ],
  )
  def sc_agg(table_h, src_h, dst_h, iota_h, summed0_out, summed1_out,
             recip_out, sbuf, dbuf, rows_v, deg_v, iota_v, dacc_v, accum,
             sdeg, g0, g1, s0, s1, sem_i):
    sem_g = [g0, g1]
    sem_s = [s0, s1]
    c = lax.axis_index("c")
    s = lax.axis_index("s")
    zeros16 = jnp.zeros((LANES,), jnp.float32)
    ones16 = jnp.ones((LANES,), jnp.float32)

    pltpu.sync_copy(iota_h, iota_v)

    def idx_pf(g):
      base = s * NCHUNK + g * GRP
      return (pltpu.async_copy(src_h.at[pl.ds(base, GRP)], sbuf.at[g % 2],
                               sem_i),
              pltpu.async_copy(dst_h.at[c, pl.ds(base, GRP)], dbuf.at[g % 2],
                               sem_i))

    def gather(t):
      g, j = divmod(t, GRP)
      return pltpu.async_copy(table_h.at[dbuf.at[g % 2, j]],
                              rows_v.at[t % DEPTH], sem_g[t % DEPTH])

    def scatter(t):
      g, j = divmod(t, GRP)
      return pltpu.async_copy(rows_v.at[t % DEPTH],
                              accum.at[sbuf.at[g % 2, j]], sem_s[t % DEPTH],
                              add=True)

    pf = idx_pf(0)

    def zero_deg(i, _):
      deg_v[i // (HALF // LANES),
            pl.ds((i % (HALF // LANES)) * LANES, LANES)] = zeros16
      return 0
    lax.fori_loop(0, DEG_ROWS * (HALF // LANES), zero_deg, 0, unroll=8)

    pf[0].wait()
    pf[1].wait()
    gathers = {}
    scatters = {}
    for x in range(AHEAD):
      gathers[x] = gather(x)

    for k in range(ROWS_PER_TILE // DEG_ROWS):
      pltpu.sync_copy(deg_v,
                      accum.at[pl.ds(s * ROWS_PER_TILE + k * DEG_ROWS,
                                     DEG_ROWS)])
    @pl.when(s == 0)
    def _():
      pltpu.sync_copy(deg_v, sdeg)
    plsc.subcore_barrier()

    for t in range(NCHUNK):
      g, j = divmod(t, GRP)
      x = t + AHEAD
      if x < NCHUNK and x >= DEPTH:
        scatters[x - DEPTH].wait()
      if j == 0 and g + 1 < NGRP:
        pf = idx_pf(g + 1)
      if j == GRP - AHEAD and g + 1 < NGRP:
        pf[0].wait()
        pf[1].wait()
      if x < NCHUNK:
        gathers[x] = gather(x)
      gathers[t].wait()
      scatters[t] = scatter(t)
      if j == GRP - 1:
        @pl.when(c == 0)
        def _(gp=g % 2):
          def deg_body(i, _):
            idx = sbuf[gp, i // (CHUNK // LANES),
                       pl.ds((i % (CHUNK // LANES)) * LANES, LANES)]
            row = lax.shift_right_logical(idx, 7)
            col = lax.bitwise_and(idx, HALF - 1)
            plsc.addupdate_scatter(deg_v, [row, col], ones16)
            return 0
          lax.fori_loop(0, GRP * (CHUNK // LANES), deg_body, 0, unroll=4)
    for t in range(NCHUNK - DEPTH, NCHUNK):
      scatters[t].wait()

    @pl.when(c == 0)
    def _():
      pltpu.sync_copy(deg_v, sdeg.at[iota_v], add=True)
    plsc.subcore_barrier()

    @pl.when(c == 0)
    def _():
      pltpu.sync_copy(accum.at[pl.ds(s * ROWS_PER_TILE, ROWS_PER_TILE)],
                      summed0_out.at[pl.ds(s * ROWS_PER_TILE, ROWS_PER_TILE)])
    @pl.when(c == 1)
    def _():
      pltpu.sync_copy(accum.at[pl.ds(s * ROWS_PER_TILE, ROWS_PER_TILE)],
                      summed1_out.at[pl.ds(s * ROWS_PER_TILE, ROWS_PER_TILE)])

    @pl.when(jnp.logical_and(c == 0, s < RECIP_TILES))
    def _():
      pltpu.sync_copy(sdeg.at[pl.ds(s * RECIP_ROWS, RECIP_ROWS)], dacc_v)
      def recip_body(i, _):
        r = i // (HALF // LANES)
        k = (i % (HALF // LANES)) * LANES
        d = dacc_v[r, pl.ds(k, LANES)]
        dacc_v[r, pl.ds(k, LANES)] = 1.0 / jnp.maximum(d, 1.0)
        return 0
      lax.fori_loop(0, RECIP_ROWS * (HALF // LANES), recip_body, 0, unroll=8)
      pltpu.sync_copy(dacc_v, recip_out.at[pl.ds(s * RECIP_ROWS, RECIP_ROWS)])

  return sc_agg(table, src2, dst3, iota)


ROW_BLK = 5000
GRID = N_NODES // ROW_BLK


def _tc_body(h_ref, ws_ref, wn_ref, bs_ref, bn_ref, s0_ref, s1_ref, r_ref,
             o_ref):
  r = r_ref[...]
  n0 = s0_ref[...] * r
  n1 = s1_ref[...] * r
  dn = (((1,), (1,)), ((), ()))
  acc = lax.dot_general(h_ref[...], ws_ref[...], dn,
                        preferred_element_type=jnp.float32)
  acc = acc + lax.dot_general(n0, wn_ref[:, 0:HALF], dn,
                              preferred_element_type=jnp.float32)
  acc = acc + lax.dot_general(n1, wn_ref[:, HALF:DIM], dn,
                              preferred_element_type=jnp.float32)
  o_ref[...] = jnp.maximum(acc + bs_ref[...] + bn_ref[...], 0.0)


def _tc_combine(h, W_self, W_neigh, b_self, b_neigh, s0, s1, recip):
  return pl.pallas_call(
      _tc_body,
      grid=(GRID,),
      in_specs=[
          pl.BlockSpec((ROW_BLK, DIM), lambda i: (i, 0)),
          pl.BlockSpec((DIM, DIM), lambda i: (0, 0)),
          pl.BlockSpec((DIM, DIM), lambda i: (0, 0)),
          pl.BlockSpec((1, DIM), lambda i: (0, 0)),
          pl.BlockSpec((1, DIM), lambda i: (0, 0)),
          pl.BlockSpec((ROW_BLK, HALF), lambda i: (i, 0)),
          pl.BlockSpec((ROW_BLK, HALF), lambda i: (i, 0)),
          pl.BlockSpec((ROW_BLK, 1), lambda i: (i, 0)),
      ],
      out_specs=pl.BlockSpec((ROW_BLK, DIM), lambda i: (i, 0)),
      out_shape=jax.ShapeDtypeStruct((N_NODES, DIM), jnp.float32),
  )(h, W_self, W_neigh, b_self, b_neigh, s0, s1, recip)


def kernel(embed_weight, W_self, b_self, W_neigh, b_neigh, edge_index):
  h = embed_weight.astype(jnp.float32)
  src = edge_index[0].astype(jnp.int32)
  dst = edge_index[1].astype(jnp.int32)

  e_pad = NS * EPT
  npad_rows = NPAD - N_NODES
  pad_src = N_NODES + jnp.arange(e_pad - N_EDGES, dtype=jnp.int32) % npad_rows
  pad_dst = jnp.arange(e_pad - N_EDGES, dtype=jnp.int32) % npad_rows
  src2 = jnp.concatenate([src, pad_src]).reshape(NS * NCHUNK, CHUNK)
  dstp = jnp.concatenate([dst, pad_dst]).reshape(NS * NCHUNK, CHUNK)
  table = jnp.concatenate([h[:, :HALF], h[:, HALF:]], axis=0)
  dst3 = jnp.stack([dstp, dstp + N_NODES])
  iota = jnp.arange(DEG_ROWS, dtype=jnp.int32)

  s0, s1, recip = _sc_aggregate(table, src2, dst3, iota)
  return _tc_combine(h, W_self, W_neigh, b_self.reshape(1, DIM),
                     b_neigh.reshape(1, DIM), s0, s1,
                     recip.reshape(NPAD, 1))
